# baseline (speedup 1.0000x reference)
import functools

import jax
import jax.numpy as jnp
from jax import lax
from jax.experimental import pallas as pl
from jax.experimental.pallas import tpu as pltpu
from jax.experimental.pallas import tpu_sc as plsc

_EMB = 64
_NC = 2
_NS = 16
_NW = _NC * _NS

_CHUNK = 640
_SUB = 128
_NSUB = _CHUNK // _SUB

_NBT = 64


def _sc_gather(t_flat, table, n_rows):
    b_per_w = n_rows // _NW
    n_chunks = b_per_w // _CHUNK
    n_pairs = n_chunks // 2

    mesh = plsc.VectorSubcoreMesh(core_axis_name="c", subcore_axis_name="s")

    @functools.partial(
        pl.kernel,
        mesh=mesh,
        out_type=jax.ShapeDtypeStruct((n_rows, _EMB), jnp.float32),
        scratch_types=[
            pltpu.VMEM((b_per_w,), jnp.int32),
            pltpu.VMEM((_CHUNK, _EMB), jnp.float32),
            pltpu.VMEM((_CHUNK, _EMB), jnp.float32),
            pltpu.SemaphoreType.DMA,
            pltpu.SemaphoreType.DMA,
            pltpu.SemaphoreType.DMA,
            pltpu.SemaphoreType.DMA,
        ],
        compiler_params=pltpu.CompilerParams(use_tc_tiling_on_sc=False),
    )
    def k(t_hbm, table_hbm, out_hbm, idx_v, rows0, rows1, gs0, gs1, os0, os1):
        wid = lax.axis_index("s") * _NC + lax.axis_index("c")
        base = wid * b_per_w

        pltpu.sync_copy(t_hbm.at[pl.ds(base, b_per_w)], idx_v)

        def fire_gather(c, rows, sem):
            for j in range(_NSUB):
                pltpu.async_copy(
                    table_hbm.at[idx_v.at[pl.ds(c * _CHUNK + j * _SUB, _SUB)]],
                    rows.at[pl.ds(j * _SUB, _SUB)],
                    sem)

        def wait_gather(rows, sem):
            pltpu.make_async_copy(
                table_hbm.at[idx_v.at[pl.ds(0, _SUB)]],
                rows, sem).wait()

        def fire_wb(c, rows, sem):
            pltpu.async_copy(
                rows, out_hbm.at[pl.ds(base + c * _CHUNK, _CHUNK)], sem)

        def wait_wb(rows, sem):
            pltpu.make_async_copy(
                rows, out_hbm.at[pl.ds(0, _CHUNK)], sem).wait()

        fire_gather(0, rows0, gs0)
        fire_gather(1, rows1, gs1)

        def body(i, carry):
            c0 = 2 * i
            wait_gather(rows0, gs0)
            fire_wb(c0, rows0, os0)
            wait_gather(rows1, gs1)
            fire_wb(c0 + 1, rows1, os1)
            wait_wb(rows0, os0)
            fire_gather(c0 + 2, rows0, gs0)
            wait_wb(rows1, os1)
            fire_gather(c0 + 3, rows1, gs1)
            return carry

        lax.fori_loop(0, n_pairs - 1, body, 0)

        c_last = n_chunks - 2
        wait_gather(rows0, gs0)
        fire_wb(c_last, rows0, os0)
        wait_gather(rows1, gs1)
        fire_wb(c_last + 1, rows1, os1)
        wait_wb(rows0, os0)
        wait_wb(rows1, os1)

    return k(t_flat, table)


def kernel(t, pos_embeddings):
    b, s = t.shape
    flat = _sc_gather(t.reshape(-1), pos_embeddings, b * s)
    one = lax.optimization_barrier(jnp.float32(1.0))
    return flat.reshape(b, s, _EMB) * one


# speedup vs baseline: 1.8648x; 1.8648x over previous
import functools

import jax
import jax.numpy as jnp
from jax import lax
from jax.experimental import pallas as pl
from jax.experimental.pallas import tpu as pltpu
from jax.experimental.pallas import tpu_sc as plsc

_EMB = 64
_SEQ = 50
_NC = 2
_NS = 16
_NW = _NC * _NS

_NB = 16


def _sc_gather(t, table, n_batch):
    b_per_w = n_batch // _NW
    n_chunks = b_per_w // _NB
    n_pairs = n_chunks // 2
    rows_per_chunk = _NB * _SEQ

    mesh = plsc.VectorSubcoreMesh(core_axis_name="c", subcore_axis_name="s")

    @functools.partial(
        pl.kernel,
        mesh=mesh,
        out_type=jax.ShapeDtypeStruct((n_batch * _SEQ, _EMB), jnp.float32),
        scratch_types=[
            pltpu.VMEM((b_per_w, _SEQ), jnp.int32),
            pltpu.VMEM((rows_per_chunk, _EMB), jnp.float32),
            pltpu.VMEM((rows_per_chunk, _EMB), jnp.float32),
            pltpu.SemaphoreType.DMA,
            pltpu.SemaphoreType.DMA,
            pltpu.SemaphoreType.DMA,
            pltpu.SemaphoreType.DMA,
        ],
        compiler_params=pltpu.CompilerParams(use_tc_tiling_on_sc=False),
    )
    def k(t_hbm, table_hbm, out_hbm, idx_v, rows0, rows1, gs0, gs1, os0, os1):
        wid = lax.axis_index("s") * _NC + lax.axis_index("c")
        base_b = wid * b_per_w
        base_r = base_b * _SEQ

        pltpu.sync_copy(t_hbm.at[pl.ds(base_b, b_per_w)], idx_v)

        def fire_gather(c, rows, sem):
            for j in range(_NB):
                pltpu.async_copy(
                    table_hbm.at[idx_v.at[c * _NB + j]],
                    rows.at[pl.ds(j * _SEQ, _SEQ)],
                    sem)

        def wait_gather(rows, sem):
            pltpu.make_async_copy(
                table_hbm.at[idx_v.at[0]], rows, sem).wait()

        def fire_wb(c, rows, sem):
            pltpu.async_copy(
                rows,
                out_hbm.at[pl.ds(base_r + c * rows_per_chunk, rows_per_chunk)],
                sem)

        def wait_wb(rows, sem):
            pltpu.make_async_copy(
                rows, out_hbm.at[pl.ds(0, rows_per_chunk)], sem).wait()

        fire_gather(0, rows0, gs0)
        fire_gather(1, rows1, gs1)

        def body(i, carry):
            c0 = 2 * i
            wait_gather(rows0, gs0)
            fire_wb(c0, rows0, os0)
            wait_gather(rows1, gs1)
            fire_wb(c0 + 1, rows1, os1)
            wait_wb(rows0, os0)
            fire_gather(c0 + 2, rows0, gs0)
            wait_wb(rows1, os1)
            fire_gather(c0 + 3, rows1, gs1)
            return carry

        lax.fori_loop(0, n_pairs - 1, body, 0)

        c_last = n_chunks - 2
        wait_gather(rows0, gs0)
        fire_wb(c_last, rows0, os0)
        wait_gather(rows1, gs1)
        fire_wb(c_last + 1, rows1, os1)
        wait_wb(rows0, os0)
        wait_wb(rows1, os1)

    return k(t, table)


def kernel(t, pos_embeddings):
    b, s = t.shape
    flat = _sc_gather(t, pos_embeddings, b)
    return flat.reshape(b, s, _EMB)


# speedup vs baseline: 1.8988x; 1.0182x over previous
import functools

import jax
import jax.numpy as jnp
from jax import lax
from jax.experimental import pallas as pl
from jax.experimental.pallas import tpu as pltpu
from jax.experimental.pallas import tpu_sc as plsc

_EMB = 64
_NC = 2
_NS = 16
_NW = _NC * _NS

_CHUNK = 640
_SUB = 128
_NSUB = _CHUNK // _SUB

_NBT = 64


def _sc_gather(t_flat, table, n_rows):
    b_per_w = n_rows // _NW
    n_chunks = b_per_w // _CHUNK
    n_pairs = n_chunks // 2

    mesh = plsc.VectorSubcoreMesh(core_axis_name="c", subcore_axis_name="s")

    @functools.partial(
        pl.kernel,
        mesh=mesh,
        out_type=jax.ShapeDtypeStruct((n_rows, _EMB), jnp.float32),
        scratch_types=[
            pltpu.VMEM((b_per_w,), jnp.int32),
            pltpu.VMEM((_CHUNK, _EMB), jnp.float32),
            pltpu.VMEM((_CHUNK, _EMB), jnp.float32),
            pltpu.SemaphoreType.DMA,
            pltpu.SemaphoreType.DMA,
            pltpu.SemaphoreType.DMA,
            pltpu.SemaphoreType.DMA,
        ],
        compiler_params=pltpu.CompilerParams(use_tc_tiling_on_sc=False),
    )
    def k(t_hbm, table_hbm, out_hbm, idx_v, rows0, rows1, gs0, gs1, os0, os1):
        wid = lax.axis_index("s") * _NC + lax.axis_index("c")
        base = wid * b_per_w

        pltpu.sync_copy(t_hbm.at[pl.ds(base, b_per_w)], idx_v)

        def fire_gather(c, rows, sem):
            for j in range(_NSUB):
                pltpu.async_copy(
                    table_hbm.at[idx_v.at[pl.ds(c * _CHUNK + j * _SUB, _SUB)]],
                    rows.at[pl.ds(j * _SUB, _SUB)],
                    sem)

        def wait_gather(rows, sem):
            pltpu.make_async_copy(
                table_hbm.at[idx_v.at[pl.ds(0, _SUB)]],
                rows, sem).wait()

        def fire_wb(c, rows, sem):
            pltpu.async_copy(
                rows, out_hbm.at[pl.ds(base + c * _CHUNK, _CHUNK)], sem)

        def wait_wb(rows, sem):
            pltpu.make_async_copy(
                rows, out_hbm.at[pl.ds(0, _CHUNK)], sem).wait()

        fire_gather(0, rows0, gs0)
        fire_gather(1, rows1, gs1)

        def body(i, carry):
            c0 = 2 * i
            wait_gather(rows0, gs0)
            fire_wb(c0, rows0, os0)
            wait_gather(rows1, gs1)
            fire_wb(c0 + 1, rows1, os1)
            wait_wb(rows0, os0)
            fire_gather(c0 + 2, rows0, gs0)
            wait_wb(rows1, os1)
            fire_gather(c0 + 3, rows1, gs1)
            return carry

        lax.fori_loop(0, n_pairs - 1, body, 0)

        c_last = n_chunks - 2
        wait_gather(rows0, gs0)
        fire_wb(c_last, rows0, os0)
        wait_gather(rows1, gs1)
        fire_wb(c_last + 1, rows1, os1)
        wait_wb(rows0, os0)
        wait_wb(rows1, os1)

    return k(t_flat, table)


def kernel(t, pos_embeddings):
    b, s = t.shape
    flat = _sc_gather(t.reshape(-1), pos_embeddings, b * s)
    return flat.reshape(b, s, _EMB)


# speedup vs baseline: 1.9127x; 1.0073x over previous
import functools

import jax
import jax.numpy as jnp
from jax import lax
from jax.experimental import pallas as pl
from jax.experimental.pallas import tpu as pltpu
from jax.experimental.pallas import tpu_sc as plsc

_EMB = 64
_NC = 2
_NS = 16
_NW = _NC * _NS

_CHUNK = 256
_SUB = 128
_NSUB = _CHUNK // _SUB
_NBUF = 4


def _sc_gather(t_flat, table, n_rows):
    b_per_w = n_rows // _NW
    n_chunks = b_per_w // _CHUNK
    n_quads = n_chunks // _NBUF

    mesh = plsc.VectorSubcoreMesh(core_axis_name="c", subcore_axis_name="s")

    @functools.partial(
        pl.kernel,
        mesh=mesh,
        out_type=jax.ShapeDtypeStruct((n_rows, _EMB), jnp.float32),
        scratch_types=[
            pltpu.VMEM((b_per_w,), jnp.int32),
        ] + [pltpu.VMEM((_CHUNK, _EMB), jnp.float32)] * _NBUF
          + [pltpu.SemaphoreType.DMA] * (2 * _NBUF),
        compiler_params=pltpu.CompilerParams(use_tc_tiling_on_sc=False),
    )
    def k(t_hbm, table_hbm, out_hbm, idx_v, *bufs_and_sems):
        rows = bufs_and_sems[:_NBUF]
        gs = bufs_and_sems[_NBUF:2 * _NBUF]
        os_ = bufs_and_sems[2 * _NBUF:3 * _NBUF]

        wid = lax.axis_index("s") * _NC + lax.axis_index("c")
        base = wid * b_per_w

        pltpu.sync_copy(t_hbm.at[pl.ds(base, b_per_w)], idx_v)

        def fire_gather(c, b):
            for j in range(_NSUB):
                pltpu.async_copy(
                    table_hbm.at[idx_v.at[pl.ds(c * _CHUNK + j * _SUB, _SUB)]],
                    rows[b].at[pl.ds(j * _SUB, _SUB)],
                    gs[b])

        def wait_gather(b):
            pltpu.make_async_copy(
                table_hbm.at[idx_v.at[pl.ds(0, _SUB)]],
                rows[b], gs[b]).wait()

        def fire_wb(c, b):
            pltpu.async_copy(
                rows[b], out_hbm.at[pl.ds(base + c * _CHUNK, _CHUNK)], os_[b])

        def wait_wb(b):
            pltpu.make_async_copy(
                rows[b], out_hbm.at[pl.ds(0, _CHUNK)], os_[b]).wait()

        for b in range(_NBUF):
            fire_gather(b, b)

        def body(i, carry):
            c0 = _NBUF * i
            for b in range(_NBUF):
                wait_gather(b)
                fire_wb(c0 + b, b)
            for b in range(_NBUF):
                wait_wb(b)
                fire_gather(c0 + _NBUF + b, b)
            return carry

        lax.fori_loop(0, n_quads - 1, body, 0)

        c_last = n_chunks - _NBUF
        for b in range(_NBUF):
            wait_gather(b)
            fire_wb(c_last + b, b)
        for b in range(_NBUF):
            wait_wb(b)

    return k(t_flat, table)


def kernel(t, pos_embeddings):
    b, s = t.shape
    flat = _sc_gather(t.reshape(-1), pos_embeddings, b * s)
    return flat.reshape(b, s, _EMB)
